# Initial kernel scaffold; baseline (speedup 1.0000x reference)
import functools

import jax
import jax.numpy as jnp
from jax import lax
from jax.experimental import pallas as pl
from jax.experimental.pallas import tpu as pltpu
from jax.experimental.pallas import tpu_sc as plsc

_NC = 2
_NS = 16
_NW = _NC * _NS
_L = 16

_B = 1280


def _gather6(T, ev_idx, val_idx, *, n_rows, n_ev, n_val, emb, n_cols):
    rows_per_w = n_rows // _NW
    n_chunks = rows_per_w // _B
    ipr = n_cols + 1
    groups = _B // _L
    n_streams = (_B * ipr) // 128

    mesh = plsc.VectorSubcoreMesh(
        core_axis_name="c", subcore_axis_name="s",
        num_cores=_NC, num_subcores=_NS)

    @functools.partial(
        pl.kernel,
        out_type=jax.ShapeDtypeStruct((n_rows * ipr, emb), jnp.float32),
        mesh=mesh,
        scratch_types=[
            pltpu.VMEM((_B,), jnp.int32),
            pltpu.VMEM((_B, n_cols), jnp.int32),
            pltpu.VMEM((_B * ipr,), jnp.int32),
            pltpu.VMEM((_B * ipr, emb), jnp.float32),
            pltpu.SemaphoreType.DMA,
        ],
    )
    def k(T_hbm, ev_hbm, vi_hbm, out_hbm, ev_v, vi_v, idx_v, rows_v, sem):
        wid = lax.axis_index("s") * _NC + lax.axis_index("c")
        base = wid * rows_per_w

        def chunk(t, carry):
            rbase = base + t * _B
            pltpu.sync_copy(ev_hbm.at[pl.ds(rbase, _B)], ev_v)
            pltpu.sync_copy(vi_hbm.at[pl.ds(rbase, _B)], vi_v)

            def grp(g, carry2):
                r16 = g * _L + lax.iota(jnp.int32, _L)
                d = r16 * ipr
                ev = ev_v[pl.ds(g * _L, _L)]
                plsc.store_scatter(idx_v, [d], ev)
                for c in range(n_cols):
                    col = jnp.full((_L,), c, jnp.int32)
                    iv = plsc.load_gather(vi_v, [r16, col])
                    plsc.store_scatter(idx_v, [d + (c + 1)],
                                       iv + (n_ev + c * n_val))
                return carry2

            lax.fori_loop(0, groups, grp, 0)

            def fire(j, carry2):
                cp = pltpu.make_async_copy(
                    T_hbm.at[idx_v.at[pl.ds(j * 128, 128)]],
                    rows_v.at[pl.ds(j * 128, 128)], sem)
                cp.start()
                return carry2

            lax.fori_loop(0, n_streams, fire, 0)

            def drain(j, carry2):
                pltpu.make_async_copy(
                    T_hbm.at[idx_v.at[pl.ds(j * 128, 128)]],
                    rows_v.at[pl.ds(j * 128, 128)], sem).wait()
                return carry2

            lax.fori_loop(0, n_streams, drain, 0)

            pltpu.sync_copy(rows_v, out_hbm.at[pl.ds(rbase * ipr, _B * ipr)])
            return carry

        lax.fori_loop(0, n_chunks, chunk, 0)

    return k(T, ev_idx, val_idx)


def kernel(event_idx, value_idx, event_table, value_tables):
    n_rows = event_idx.shape[0]
    n_cols, n_val, emb = value_tables.shape
    n_ev = event_table.shape[0]
    T = jnp.concatenate(
        [event_table, value_tables.reshape(n_cols * n_val, emb)], axis=0)
    out6 = _gather6(
        T,
        event_idx.astype(jnp.int32),
        value_idx.astype(jnp.int32),
        n_rows=n_rows, n_ev=n_ev, n_val=n_val, emb=emb, n_cols=n_cols)
    return out6.reshape(n_rows, (n_cols + 1) * emb)


# speedup vs baseline: 30.9915x; 30.9915x over previous
import functools

import jax
import jax.numpy as jnp
from jax import lax
from jax.experimental import pallas as pl
from jax.experimental.pallas import tpu as pltpu
from jax.experimental.pallas import tpu_sc as plsc

_NC = 2
_NS = 16
_NW = _NC * _NS
_L = 16

_B = 1280
_EVW = 16


def _lookup(ev16, vt_flat, ev_idx, val_idx, *, n_rows, n_val, emb, n_cols):
    rows_per_w = n_rows // _NW
    n_chunks = rows_per_w // _B
    groups = _B // _L
    n_streams = _B // 128
    out_w = (n_cols + 1) * emb

    mesh = plsc.VectorSubcoreMesh(
        core_axis_name="c", subcore_axis_name="s",
        num_cores=_NC, num_subcores=_NS)

    @functools.partial(
        pl.kernel,
        out_type=jax.ShapeDtypeStruct((n_rows, out_w), jnp.float32),
        mesh=mesh,
        scratch_types=[
            pltpu.VMEM((n_cols * n_val * emb,), jnp.float32),
            pltpu.VMEM((_B,), jnp.int32),
            pltpu.VMEM((_B, n_cols), jnp.int32),
            pltpu.VMEM((_B, _EVW), jnp.float32),
            pltpu.VMEM((_B, out_w), jnp.float32),
            pltpu.SemaphoreType.DMA,
        ],
        compiler_params=pltpu.CompilerParams(
            needs_layout_passes=False, use_tc_tiling_on_sc=False),
    )
    def k(ev16_hbm, vt_hbm, ev_hbm, vi_hbm, out_hbm,
          vt_v, ev_v, vi_v, stage_v, out_v, sem):
        wid = lax.axis_index("s") * _NC + lax.axis_index("c")
        base = wid * rows_per_w
        pltpu.sync_copy(vt_hbm, vt_v)

        def chunk(t, carry):
            rbase = base + t * _B
            pltpu.sync_copy(ev_hbm.at[pl.ds(rbase, _B)], ev_v)
            pltpu.sync_copy(vi_hbm.at[pl.ds(rbase, _B)], vi_v)

            def fire(j, c2):
                pltpu.make_async_copy(
                    ev16_hbm.at[ev_v.at[pl.ds(j * 128, 128)]],
                    stage_v.at[pl.ds(j * 128, 128)], sem).start()
                return c2

            lax.fori_loop(0, n_streams, fire, 0)

            def grp_val(g, c2):
                r16 = g * _L + lax.iota(jnp.int32, _L)
                for c in range(n_cols):
                    col = jnp.full((_L,), c, jnp.int32)
                    iv = plsc.load_gather(vi_v, [r16, col])
                    iv3 = iv * emb
                    for e in range(emb):
                        x = plsc.load_gather(
                            vt_v, [iv3 + ((c * n_val) * emb + e)])
                        dcol = jnp.full((_L,), (1 + c) * emb + e, jnp.int32)
                        plsc.store_scatter(out_v, [r16, dcol], x)
                return c2

            lax.fori_loop(0, groups, grp_val, 0)

            def drain(j, c2):
                pltpu.make_async_copy(
                    ev16_hbm.at[ev_v.at[pl.ds(j * 128, 128)]],
                    stage_v.at[pl.ds(j * 128, 128)], sem).wait()
                return c2

            lax.fori_loop(0, n_streams, drain, 0)

            def grp_ev(g, c2):
                r16 = g * _L + lax.iota(jnp.int32, _L)
                for e in range(emb):
                    ecol = jnp.full((_L,), e, jnp.int32)
                    x = plsc.load_gather(stage_v, [r16, ecol])
                    plsc.store_scatter(out_v, [r16, ecol], x)
                return c2

            lax.fori_loop(0, groups, grp_ev, 0)

            pltpu.sync_copy(out_v, out_hbm.at[pl.ds(rbase, _B)])
            return carry

        lax.fori_loop(0, n_chunks, chunk, 0)

    return k(ev16, vt_flat, ev_idx, val_idx)


def kernel(event_idx, value_idx, event_table, value_tables):
    n_rows = event_idx.shape[0]
    n_cols, n_val, emb = value_tables.shape
    ev16 = jnp.pad(event_table, ((0, 0), (0, _EVW - emb)))
    vt_flat = value_tables.reshape(n_cols * n_val * emb)
    return _lookup(
        ev16, vt_flat,
        event_idx.astype(jnp.int32),
        value_idx.astype(jnp.int32),
        n_rows=n_rows, n_val=n_val, emb=emb, n_cols=n_cols)
